# Initial kernel scaffold; baseline (speedup 1.0000x reference)
import functools

import jax
import jax.numpy as jnp
from jax import lax
from jax.experimental import pallas as pl
from jax.experimental.pallas import tpu as pltpu
from jax.experimental.pallas import tpu_sc as plsc

N_NODES = 100000
E_EDGES = 6400000
LANE = 128
ROWS = 784
NP = ROWS * LANE
NW = 32
K = 8
RPW = 1568
EP = NW * RPW * LANE
ROWS_E = EP // LANE
NCHUNK = RPW // K
ZSLICE = NP // 16

_mesh = plsc.VectorSubcoreMesh(core_axis_name="c", subcore_axis_name="s")


@functools.partial(
    pl.kernel, mesh=_mesh,
    out_type=jax.ShapeDtypeStruct((2, NP), jnp.float32),
    scratch_types=[
        pltpu.VMEM((K, LANE), jnp.int32),
        pltpu.VMEM((LANE,), jnp.float32),
        pltpu.VMEM_SHARED((NP,), jnp.float32),
    ],
)
def _sc_degree(dst_hbm, zeros_hbm, out_hbm, dst_v, ones_v, z_sh):
    cid = lax.axis_index("c")
    sid = lax.axis_index("s")
    wid = sid * 2 + cid
    pltpu.sync_copy(zeros_hbm.at[pl.ds(sid * ZSLICE, ZSLICE)],
                    z_sh.at[pl.ds(sid * ZSLICE, ZSLICE)])
    for t in range(LANE // 16):
        ones_v[pl.ds(t * 16, 16)] = jnp.ones((16,), jnp.float32)
    plsc.subcore_barrier()

    def body(ci, carry):
        rowbase = wid * RPW + ci * K
        pltpu.sync_copy(dst_hbm.at[pl.ds(rowbase, K)], dst_v)
        for j in range(K):
            pltpu.sync_copy(ones_v, z_sh.at[dst_v.at[j]], add=True)
        return carry

    lax.fori_loop(0, NCHUNK, body, 0)
    plsc.subcore_barrier()
    pltpu.sync_copy(z_sh.at[pl.ds(sid * ZSLICE, ZSLICE)],
                    out_hbm.at[cid, pl.ds(sid * ZSLICE, ZSLICE)])


def _make_prop_cols(ntab):
    scratch = [
        pltpu.VMEM((K, LANE), jnp.int32),
        pltpu.VMEM((K, LANE), jnp.int32),
    ]
    scratch += [pltpu.VMEM((K, LANE), jnp.float32) for _ in range(ntab)]
    scratch += [pltpu.VMEM_SHARED((NP,), jnp.float32) for _ in range(ntab)]
    scratch.append(pltpu.SemaphoreType.DMA)

    @functools.partial(
        pl.kernel, mesh=_mesh,
        out_type=[jax.ShapeDtypeStruct((2, NP), jnp.float32)
                  for _ in range(ntab)],
        scratch_types=scratch,
    )
    def prop(src_hbm, dst_hbm, zeros_hbm, *args):
        tabs = args[:ntab]
        outs = args[ntab:2 * ntab]
        src_v, dst_v = args[2 * ntab:2 * ntab + 2]
        msgs = args[2 * ntab + 2:3 * ntab + 2]
        zs = args[3 * ntab + 2:4 * ntab + 2]
        sem = args[4 * ntab + 2]
        cid = lax.axis_index("c")
        sid = lax.axis_index("s")
        wid = sid * 2 + cid
        for t in range(ntab):
            pltpu.sync_copy(zeros_hbm.at[pl.ds(sid * ZSLICE, ZSLICE)],
                            zs[t].at[pl.ds(sid * ZSLICE, ZSLICE)])
        plsc.subcore_barrier()

        def body(ci, carry):
            rowbase = wid * RPW + ci * K
            pltpu.sync_copy(src_hbm.at[pl.ds(rowbase, K)], src_v)
            pltpu.sync_copy(dst_hbm.at[pl.ds(rowbase, K)], dst_v)
            handles = []
            for t in range(ntab):
                for j in range(K):
                    handles.append(pltpu.async_copy(
                        tabs[t].at[src_v.at[j]], msgs[t].at[j], sem))
            for h in handles:
                h.wait()
            for t in range(ntab):
                for j in range(K):
                    pltpu.sync_copy(msgs[t].at[j],
                                    zs[t].at[dst_v.at[j]], add=True)
            return carry

        lax.fori_loop(0, NCHUNK, body, 0)
        plsc.subcore_barrier()
        for t in range(ntab):
            pltpu.sync_copy(zs[t].at[pl.ds(sid * ZSLICE, ZSLICE)],
                            outs[t].at[cid, pl.ds(sid * ZSLICE, ZSLICE)])

    return prop


_sc_prop2 = _make_prop_cols(2)
_sc_prop1 = _make_prop_cols(1)


@functools.partial(
    pl.kernel, mesh=_mesh,
    out_type=jax.ShapeDtypeStruct((2, NP, 16), jnp.float32),
    scratch_types=[
        pltpu.VMEM((K, LANE), jnp.int32),
        pltpu.VMEM((K, LANE), jnp.int32),
        pltpu.VMEM((K, LANE, 16), jnp.float32),
        pltpu.VMEM_SHARED((NP, 16), jnp.float32),
        pltpu.SemaphoreType.DMA,
    ],
)
def _sc_prop16(src_hbm, dst_hbm, zeros_hbm, tab_hbm, out_hbm,
               src_v, dst_v, msg_v, z_sh, sem):
    cid = lax.axis_index("c")
    sid = lax.axis_index("s")
    wid = sid * 2 + cid
    pltpu.sync_copy(zeros_hbm.at[pl.ds(sid * ZSLICE, ZSLICE)],
                    z_sh.at[pl.ds(sid * ZSLICE, ZSLICE)])
    plsc.subcore_barrier()

    def body(ci, carry):
        rowbase = wid * RPW + ci * K
        pltpu.sync_copy(src_hbm.at[pl.ds(rowbase, K)], src_v)
        pltpu.sync_copy(dst_hbm.at[pl.ds(rowbase, K)], dst_v)
        handles = [pltpu.async_copy(tab_hbm.at[src_v.at[j]],
                                    msg_v.at[j], sem) for j in range(K)]
        for h in handles:
            h.wait()
        for j in range(K):
            pltpu.sync_copy(msg_v.at[j], z_sh.at[dst_v.at[j]], add=True)
        return carry

    lax.fori_loop(0, NCHUNK, body, 0)
    plsc.subcore_barrier()
    pltpu.sync_copy(z_sh.at[pl.ds(sid * ZSLICE, ZSLICE)],
                    out_hbm.at[cid, pl.ds(sid * ZSLICE, ZSLICE)])


_BR = 56
_GRID = ROWS // _BR


def _rows_spec(nplanes):
    if nplanes == 1:
        return pl.BlockSpec((_BR, LANE), lambda i: (i, 0))
    return pl.BlockSpec((nplanes, _BR, LANE), lambda i: (0, i, 0))


def _smem_spec():
    return pl.BlockSpec(memory_space=pltpu.SMEM)


def _tc_b_body(dega_ref, degb_ref, x0_ref, x1_ref,
               dinv_ref, y10_ref, y11_ref):
    deg = dega_ref[...] + degb_ref[...] + 1.0
    dinv = 1.0 / jnp.sqrt(deg)
    dinv_ref[...] = dinv
    y10_ref[...] = dinv * x0_ref[...]
    y11_ref[...] = dinv * x1_ref[...]


def _tc_d1_body(z1a_ref, z1b_ref, y1_ref, dinv_ref, w1_ref, b1_ref,
                w2_ref, y2_ref):
    dinv = dinv_ref[...]
    q = [dinv * (z1a_ref[g] + z1b_ref[g] + y1_ref[g]) for g in range(2)]
    h1 = []
    for f in range(32):
        acc = q[0] * w1_ref[0, f] + q[1] * w1_ref[1, f] + b1_ref[f]
        h1.append(jnp.maximum(acc, 0.0))
    for g in range(16):
        acc = h1[0] * w2_ref[0, g]
        for f in range(1, 32):
            acc = acc + h1[f] * w2_ref[f, g]
        y2_ref[g] = dinv * acc


def _tc_d2_body(z2a_ref, z2b_ref, y2_ref, dinv_ref, b2_ref, w3_ref,
                y3_ref, hsum_ref):
    i = pl.program_id(0)
    dinv = dinv_ref[...]
    sub = lax.broadcasted_iota(jnp.int32, (_BR, LANE), 0)
    lane = lax.broadcasted_iota(jnp.int32, (_BR, LANE), 1)
    gid = (i * _BR + sub) * LANE + lane
    mask = (gid < N_NODES).astype(jnp.float32)

    @pl.when(i == 0)
    def _init():
        hsum_ref[...] = jnp.zeros((16, 1, LANE), jnp.float32)

    acc3 = None
    for f in range(16):
        h2 = jnp.maximum(
            dinv * (z2a_ref[f] + z2b_ref[f] + y2_ref[f]) + b2_ref[f], 0.0)
        hm = h2 * mask
        hsum_ref[f] = hsum_ref[f] + jnp.sum(hm, axis=0, keepdims=True)
        term = h2 * w3_ref[f, 0]
        acc3 = term if acc3 is None else acc3 + term
    y3_ref[...] = dinv * acc3


def _tc_e_body(z3a_ref, z3b_ref, y3_ref, dinv_ref, b3_ref,
               hsum_ref, wf_ref, bf_ref, choice_ref, value_ref):
    c = dinv_ref[...] * (z3a_ref[...] + z3b_ref[...] + y3_ref[...]) + b3_ref[0]
    sub = lax.broadcasted_iota(jnp.int32, (ROWS, LANE), 0)
    lane = lax.broadcasted_iota(jnp.int32, (ROWS, LANE), 1)
    valid = (sub * LANE + lane) < N_NODES
    cm = jnp.where(valid, c, -jnp.inf)
    m = jnp.max(cm)
    e = jnp.where(valid, jnp.exp(c - m), 0.0)
    s = jnp.sum(e)
    choice_ref[...] = e / s
    val = bf_ref[0]
    for f in range(16):
        val = val + jnp.sum(hsum_ref[f]) / float(N_NODES) * wf_ref[f, 0]
    value_ref[...] = jnp.full((1, LANE), val, jnp.float32)


def _tc_b(dega, degb, x0, x1):
    return pl.pallas_call(
        _tc_b_body,
        grid=(_GRID,),
        in_specs=[_rows_spec(1)] * 4,
        out_specs=[_rows_spec(1)] * 3,
        out_shape=[jax.ShapeDtypeStruct((ROWS, LANE), jnp.float32)] * 3,
    )(dega, degb, x0, x1)


def _tc_d1(z1a, z1b, y1, dinv, W1, b1, W2):
    return pl.pallas_call(
        _tc_d1_body,
        grid=(_GRID,),
        in_specs=[_rows_spec(2), _rows_spec(2), _rows_spec(2), _rows_spec(1),
                  _smem_spec(), _smem_spec(), _smem_spec()],
        out_specs=[_rows_spec(16)],
        out_shape=[jax.ShapeDtypeStruct((16, ROWS, LANE), jnp.float32)],
    )(z1a, z1b, y1, dinv, W1, b1, W2)[0]


def _tc_d2(z2a, z2b, y2, dinv, b2, W3):
    return pl.pallas_call(
        _tc_d2_body,
        grid=(_GRID,),
        in_specs=[_rows_spec(16), _rows_spec(16), _rows_spec(16),
                  _rows_spec(1), _smem_spec(), _smem_spec()],
        out_specs=[_rows_spec(1),
                   pl.BlockSpec((16, 1, LANE), lambda i: (0, 0, 0))],
        out_shape=[jax.ShapeDtypeStruct((ROWS, LANE), jnp.float32),
                   jax.ShapeDtypeStruct((16, 1, LANE), jnp.float32)],
    )(z2a, z2b, y2, dinv, b2, W3)


def _tc_e(z3a, z3b, y3, dinv, b3, hsum, Wf, bf):
    return pl.pallas_call(
        _tc_e_body,
        grid=(),
        in_specs=[pl.BlockSpec(memory_space=pltpu.ANY)] * 0
        + [pl.BlockSpec((ROWS, LANE), lambda: (0, 0))] * 4
        + [_smem_spec(), pl.BlockSpec((16, 1, LANE), lambda: (0, 0, 0)),
           _smem_spec(), _smem_spec()],
        out_specs=[pl.BlockSpec((ROWS, LANE), lambda: (0, 0)),
                   pl.BlockSpec((1, LANE), lambda: (0, 0))],
        out_shape=[jax.ShapeDtypeStruct((ROWS, LANE), jnp.float32),
                   jax.ShapeDtypeStruct((1, LANE), jnp.float32)],
    )(z3a, z3b, y3, dinv, b3, hsum, Wf, bf)


@jax.jit
def _run(x, edge_index, W1, b1, W2, b2, W3, b3, Wf, bf):
    f32 = jnp.float32
    pad_e = EP - E_EDGES
    ar = jnp.arange(pad_e, dtype=jnp.int32)
    src_rows = jnp.concatenate(
        [edge_index[0], ar % N_NODES]).reshape(ROWS_E, LANE)
    dst_rows = jnp.concatenate(
        [edge_index[1], N_NODES + ar % (NP - N_NODES)]).reshape(ROWS_E, LANE)
    zeros16 = jnp.zeros((NP, 16), f32)
    zeros1 = jnp.zeros((NP,), f32)

    xp = jnp.pad(x, ((0, NP - N_NODES), (0, 0)))
    x0 = xp[:, 0].reshape(ROWS, LANE)
    x1 = xp[:, 1].reshape(ROWS, LANE)

    deg = _sc_degree(dst_rows, zeros1)
    dega = deg[0].reshape(ROWS, LANE)
    degb = deg[1].reshape(ROWS, LANE)
    dinv, y10, y11 = _tc_b(dega, degb, x0, x1)

    z1 = _sc_prop2(src_rows, dst_rows, zeros1,
                   y10.reshape(NP), y11.reshape(NP))
    z1a = jnp.stack([z1[0][0], z1[1][0]]).reshape(2, ROWS, LANE)
    z1b = jnp.stack([z1[0][1], z1[1][1]]).reshape(2, ROWS, LANE)
    y1 = jnp.stack([y10, y11])
    y2 = _tc_d1(z1a, z1b, y1, dinv, W1, b1, W2)

    y2tab = y2.reshape(16, NP).T
    z2 = _sc_prop16(src_rows, dst_rows, zeros16, y2tab)
    z2p = z2.transpose(0, 2, 1).reshape(2, 16, ROWS, LANE)
    y3, hsum = _tc_d2(z2p[0], z2p[1], y2, dinv, b2, W3)

    z3 = _sc_prop1(src_rows, dst_rows, zeros1, y3.reshape(NP))[0]
    choice2d, value2d = _tc_e(
        z3[0].reshape(ROWS, LANE), z3[1].reshape(ROWS, LANE),
        y3, dinv, b3, hsum, Wf, bf)
    choice = choice2d.reshape(NP, 1)[:N_NODES]
    value = value2d[:1, :1]
    return choice, value


def kernel(x, edge_index, choices, W1, b1, W2, b2, W3, b3, Wf, bf):
    return _run(x, edge_index, W1, b1, W2, b2, W3, b3, Wf, bf)


# speedup vs baseline: 24.5173x; 24.5173x over previous
import functools

import jax
import jax.numpy as jnp
from jax import lax
from jax.experimental import pallas as pl
from jax.experimental.pallas import tpu as pltpu
from jax.experimental.pallas import tpu_sc as plsc

N_NODES = 100000
E_EDGES = 6400000
LANE = 128
ROWS = 784
NP = ROWS * LANE
NW = 32
K = 8
RPW = 1568
EP = NW * RPW * LANE
ROWS_E = EP // LANE
NCHUNK = RPW // K
ZSLICE = NP // 16

@functools.cache
def _mesh():
    return plsc.VectorSubcoreMesh(core_axis_name="c", subcore_axis_name="s")


@functools.cache
def _get_sc_degree():
    return pl.kernel(
        _sc_degree_body, mesh=_mesh(),
        out_type=jax.ShapeDtypeStruct((2, NP), jnp.float32),
        scratch_types=[
            pltpu.VMEM((K, LANE), jnp.int32),
            pltpu.VMEM((LANE,), jnp.float32),
            pltpu.VMEM_SHARED((NP,), jnp.float32),
        ],
    )


def _sc_degree_body(dst_hbm, zeros_hbm, out_hbm, dst_v, ones_v, z_sh):
    cid = lax.axis_index("c")
    sid = lax.axis_index("s")
    wid = sid * 2 + cid
    pltpu.sync_copy(zeros_hbm.at[pl.ds(sid * ZSLICE, ZSLICE)],
                    z_sh.at[pl.ds(sid * ZSLICE, ZSLICE)])
    for t in range(LANE // 16):
        ones_v[pl.ds(t * 16, 16)] = jnp.ones((16,), jnp.float32)
    plsc.subcore_barrier()

    def body(ci, carry):
        rowbase = wid * RPW + ci * K
        pltpu.sync_copy(dst_hbm.at[pl.ds(rowbase, K)], dst_v)
        for j in range(K):
            pltpu.sync_copy(ones_v, z_sh.at[dst_v.at[j]], add=True)
        return carry

    lax.fori_loop(0, NCHUNK, body, 0)
    plsc.subcore_barrier()
    pltpu.sync_copy(z_sh.at[pl.ds(sid * ZSLICE, ZSLICE)],
                    out_hbm.at[cid, pl.ds(sid * ZSLICE, ZSLICE)])


@functools.cache
def _make_prop_cols(ntab):
    scratch = [
        pltpu.VMEM((K, LANE), jnp.int32),
        pltpu.VMEM((K, LANE), jnp.int32),
    ]
    scratch += [pltpu.VMEM((K, LANE), jnp.float32) for _ in range(ntab)]
    scratch += [pltpu.VMEM_SHARED((NP,), jnp.float32) for _ in range(ntab)]
    scratch.append(pltpu.SemaphoreType.DMA)

    @functools.partial(
        pl.kernel, mesh=_mesh(),
        out_type=[jax.ShapeDtypeStruct((2, NP), jnp.float32)
                  for _ in range(ntab)],
        scratch_types=scratch,
    )
    def prop(src_hbm, dst_hbm, zeros_hbm, *args):
        tabs = args[:ntab]
        outs = args[ntab:2 * ntab]
        src_v, dst_v = args[2 * ntab:2 * ntab + 2]
        msgs = args[2 * ntab + 2:3 * ntab + 2]
        zs = args[3 * ntab + 2:4 * ntab + 2]
        sem = args[4 * ntab + 2]
        cid = lax.axis_index("c")
        sid = lax.axis_index("s")
        wid = sid * 2 + cid
        for t in range(ntab):
            pltpu.sync_copy(zeros_hbm.at[pl.ds(sid * ZSLICE, ZSLICE)],
                            zs[t].at[pl.ds(sid * ZSLICE, ZSLICE)])
        plsc.subcore_barrier()

        def body(ci, carry):
            rowbase = wid * RPW + ci * K
            pltpu.sync_copy(src_hbm.at[pl.ds(rowbase, K)], src_v)
            pltpu.sync_copy(dst_hbm.at[pl.ds(rowbase, K)], dst_v)
            handles = []
            for t in range(ntab):
                for j in range(K):
                    handles.append(pltpu.async_copy(
                        tabs[t].at[src_v.at[j]], msgs[t].at[j], sem))
            for h in handles:
                h.wait()
            for t in range(ntab):
                for j in range(K):
                    pltpu.sync_copy(msgs[t].at[j],
                                    zs[t].at[dst_v.at[j]], add=True)
            return carry

        lax.fori_loop(0, NCHUNK, body, 0)
        plsc.subcore_barrier()
        for t in range(ntab):
            pltpu.sync_copy(zs[t].at[pl.ds(sid * ZSLICE, ZSLICE)],
                            outs[t].at[cid, pl.ds(sid * ZSLICE, ZSLICE)])

    return prop


_BR = 56
_GRID = ROWS // _BR


def _rows_spec(nplanes):
    if nplanes == 1:
        return pl.BlockSpec((_BR, LANE), lambda i: (i, 0))
    return pl.BlockSpec((nplanes, _BR, LANE), lambda i: (0, i, 0))


def _smem_spec():
    return pl.BlockSpec(memory_space=pltpu.SMEM)


def _tc_b_body(dega_ref, degb_ref, x0_ref, x1_ref,
               dinv_ref, y10_ref, y11_ref):
    deg = dega_ref[...] + degb_ref[...] + 1.0
    dinv = 1.0 / jnp.sqrt(deg)
    dinv_ref[...] = dinv
    y10_ref[...] = dinv * x0_ref[...]
    y11_ref[...] = dinv * x1_ref[...]


def _tc_d1_body(z1a_ref, z1b_ref, y1_ref, dinv_ref, w1_ref, b1_ref,
                w2_ref, y2_ref):
    dinv = dinv_ref[...]
    q = [dinv * (z1a_ref[g] + z1b_ref[g] + y1_ref[g]) for g in range(2)]
    h1 = []
    for f in range(32):
        acc = q[0] * w1_ref[0, f] + q[1] * w1_ref[1, f] + b1_ref[f]
        h1.append(jnp.maximum(acc, 0.0))
    for g in range(16):
        acc = h1[0] * w2_ref[0, g]
        for f in range(1, 32):
            acc = acc + h1[f] * w2_ref[f, g]
        y2_ref[g] = dinv * acc


def _tc_d2_body(z2a_ref, z2b_ref, y2_ref, dinv_ref, b2_ref, w3_ref,
                y3_ref, hsum_ref):
    i = pl.program_id(0)
    dinv = dinv_ref[...]
    sub = lax.broadcasted_iota(jnp.int32, (_BR, LANE), 0)
    lane = lax.broadcasted_iota(jnp.int32, (_BR, LANE), 1)
    gid = (i * _BR + sub) * LANE + lane
    mask = (gid < N_NODES).astype(jnp.float32)

    @pl.when(i == 0)
    def _init():
        hsum_ref[...] = jnp.zeros((16, 1, LANE), jnp.float32)

    acc3 = None
    for f in range(16):
        h2 = jnp.maximum(
            dinv * (z2a_ref[f] + z2b_ref[f] + y2_ref[f]) + b2_ref[f], 0.0)
        hm = h2 * mask
        hsum_ref[f] = hsum_ref[f] + jnp.sum(hm, axis=0, keepdims=True)
        term = h2 * w3_ref[f, 0]
        acc3 = term if acc3 is None else acc3 + term
    y3_ref[...] = dinv * acc3


def _tc_e_body(z3a_ref, z3b_ref, y3_ref, dinv_ref, b3_ref,
               hsum_ref, wf_ref, bf_ref, choice_ref, value_ref):
    c = dinv_ref[...] * (z3a_ref[...] + z3b_ref[...] + y3_ref[...]) + b3_ref[0]
    sub = lax.broadcasted_iota(jnp.int32, (ROWS, LANE), 0)
    lane = lax.broadcasted_iota(jnp.int32, (ROWS, LANE), 1)
    valid = (sub * LANE + lane) < N_NODES
    cm = jnp.where(valid, c, -jnp.inf)
    m = jnp.max(cm)
    e = jnp.where(valid, jnp.exp(c - m), 0.0)
    s = jnp.sum(e)
    choice_ref[...] = e / s
    val = bf_ref[0]
    for f in range(16):
        val = val + jnp.sum(hsum_ref[f]) / float(N_NODES) * wf_ref[f, 0]
    value_ref[...] = jnp.full((1, LANE), val, jnp.float32)


def _tc_b(dega, degb, x0, x1):
    return pl.pallas_call(
        _tc_b_body,
        grid=(_GRID,),
        in_specs=[_rows_spec(1)] * 4,
        out_specs=[_rows_spec(1)] * 3,
        out_shape=[jax.ShapeDtypeStruct((ROWS, LANE), jnp.float32)] * 3,
    )(dega, degb, x0, x1)


def _tc_d1(z1a, z1b, y1, dinv, W1, b1, W2):
    return pl.pallas_call(
        _tc_d1_body,
        grid=(_GRID,),
        in_specs=[_rows_spec(2), _rows_spec(2), _rows_spec(2), _rows_spec(1),
                  _smem_spec(), _smem_spec(), _smem_spec()],
        out_specs=[_rows_spec(16)],
        out_shape=[jax.ShapeDtypeStruct((16, ROWS, LANE), jnp.float32)],
    )(z1a, z1b, y1, dinv, W1, b1, W2)[0]


def _tc_d2(z2a, z2b, y2, dinv, b2, W3):
    return pl.pallas_call(
        _tc_d2_body,
        grid=(_GRID,),
        in_specs=[_rows_spec(16), _rows_spec(16), _rows_spec(16),
                  _rows_spec(1), _smem_spec(), _smem_spec()],
        out_specs=[_rows_spec(1),
                   pl.BlockSpec((16, 1, LANE), lambda i: (0, 0, 0))],
        out_shape=[jax.ShapeDtypeStruct((ROWS, LANE), jnp.float32),
                   jax.ShapeDtypeStruct((16, 1, LANE), jnp.float32)],
    )(z2a, z2b, y2, dinv, b2, W3)


def _tc_e(z3a, z3b, y3, dinv, b3, hsum, Wf, bf):
    return pl.pallas_call(
        _tc_e_body,
        grid=(),
        in_specs=[pl.BlockSpec((ROWS, LANE), lambda: (0, 0))] * 4
        + [_smem_spec(), pl.BlockSpec((16, 1, LANE), lambda: (0, 0, 0)),
           _smem_spec(), _smem_spec()],
        out_specs=[pl.BlockSpec((ROWS, LANE), lambda: (0, 0)),
                   pl.BlockSpec((1, LANE), lambda: (0, 0))],
        out_shape=[jax.ShapeDtypeStruct((ROWS, LANE), jnp.float32),
                   jax.ShapeDtypeStruct((1, LANE), jnp.float32)],
    )(z3a, z3b, y3, dinv, b3, hsum, Wf, bf)


@jax.jit
def _run(x, edge_index, W1, b1, W2, b2, W3, b3, Wf, bf):
    f32 = jnp.float32
    pad_e = EP - E_EDGES
    ar = jnp.arange(pad_e, dtype=jnp.int32)
    src_rows = jnp.concatenate(
        [edge_index[0], ar % N_NODES]).reshape(ROWS_E, LANE)
    dst_rows = jnp.concatenate(
        [edge_index[1], N_NODES + ar % (NP - N_NODES)]).reshape(ROWS_E, LANE)
    zeros1 = jnp.zeros((NP,), f32)

    xp = jnp.pad(x, ((0, NP - N_NODES), (0, 0)))
    x0 = xp[:, 0].reshape(ROWS, LANE)
    x1 = xp[:, 1].reshape(ROWS, LANE)

    deg = _get_sc_degree()(dst_rows, zeros1)
    dega = deg[0].reshape(ROWS, LANE)
    degb = deg[1].reshape(ROWS, LANE)
    dinv, y10, y11 = _tc_b(dega, degb, x0, x1)

    z1 = _make_prop_cols(2)(src_rows, dst_rows, zeros1,
                            y10.reshape(NP), y11.reshape(NP))
    z1a = jnp.stack([z1[0][0], z1[1][0]]).reshape(2, ROWS, LANE)
    z1b = jnp.stack([z1[0][1], z1[1][1]]).reshape(2, ROWS, LANE)
    y1 = jnp.stack([y10, y11])
    y2 = _tc_d1(z1a, z1b, y1, dinv, W1, b1, W2)

    z2 = _make_prop_cols(16)(src_rows, dst_rows, zeros1,
                             *[y2[f].reshape(NP) for f in range(16)])
    z2p = jnp.stack([z2[f] for f in range(16)], axis=1)
    z2p = z2p.reshape(2, 16, ROWS, LANE)
    y3, hsum = _tc_d2(z2p[0], z2p[1], y2, dinv, b2, W3)

    z3 = _make_prop_cols(1)(src_rows, dst_rows, zeros1,
                            y3.reshape(NP))[0]
    choice2d, value2d = _tc_e(
        z3[0].reshape(ROWS, LANE), z3[1].reshape(ROWS, LANE),
        y3, dinv, b3, hsum, Wf, bf)
    choice = choice2d.reshape(NP, 1)[:N_NODES]
    value = value2d[:1, :1]
    return choice, value


def kernel(x, edge_index, choices, W1, b1, W2, b2, W3, b3, Wf, bf):
    return _run(x, edge_index, W1, b1, W2, b2, W3, b3, Wf, bf)


# speedup vs baseline: 31.2427x; 1.2743x over previous
import functools

import jax
import jax.numpy as jnp
from jax import lax
from jax.experimental import pallas as pl
from jax.experimental.pallas import tpu as pltpu
from jax.experimental.pallas import tpu_sc as plsc

N_NODES = 100000
E_EDGES = 6400000
LANE = 128
ROWS = 784
NP = ROWS * LANE
NW = 32
K = 8
RPW = 1568
EP = NW * RPW * LANE
ROWS_E = EP // LANE
NCHUNK = RPW // K
ZSLICE = NP // 16

@functools.cache
def _mesh():
    return plsc.VectorSubcoreMesh(core_axis_name="c", subcore_axis_name="s")


@functools.cache
def _get_sc_degree():
    return pl.kernel(
        _sc_degree_body, mesh=_mesh(),
        out_type=jax.ShapeDtypeStruct((2, NP), jnp.float32),
        scratch_types=[
            pltpu.VMEM((K, LANE), jnp.int32),
            pltpu.VMEM((LANE,), jnp.float32),
            pltpu.VMEM_SHARED((NP,), jnp.float32),
            pltpu.SemaphoreType.DMA,
        ],
    )


def _sc_degree_body(dst_hbm, zeros_hbm, out_hbm, dst_v, ones_v, z_sh, sem):
    cid = lax.axis_index("c")
    sid = lax.axis_index("s")
    wid = sid * 2 + cid
    pltpu.sync_copy(zeros_hbm.at[pl.ds(sid * ZSLICE, ZSLICE)],
                    z_sh.at[pl.ds(sid * ZSLICE, ZSLICE)])
    for t in range(LANE // 16):
        ones_v[pl.ds(t * 16, 16)] = jnp.ones((16,), jnp.float32)
    plsc.subcore_barrier()

    def body(ci, carry):
        rowbase = wid * RPW + ci * K
        pltpu.sync_copy(dst_hbm.at[pl.ds(rowbase, K)], dst_v)
        sh = [pltpu.async_copy(ones_v, z_sh.at[dst_v.at[j]], sem, add=True)
              for j in range(K)]
        for h in sh:
            h.wait()
        return carry

    lax.fori_loop(0, NCHUNK, body, 0)
    plsc.subcore_barrier()
    pltpu.sync_copy(z_sh.at[pl.ds(sid * ZSLICE, ZSLICE)],
                    out_hbm.at[cid, pl.ds(sid * ZSLICE, ZSLICE)])


@functools.cache
def _make_prop_cols(ntab):
    scratch = [
        pltpu.VMEM((K, LANE), jnp.int32),
        pltpu.VMEM((K, LANE), jnp.int32),
    ]
    scratch += [pltpu.VMEM((K, LANE), jnp.float32) for _ in range(ntab)]
    scratch += [pltpu.VMEM_SHARED((NP,), jnp.float32) for _ in range(ntab)]
    scratch.append(pltpu.SemaphoreType.DMA)
    scratch.append(pltpu.SemaphoreType.DMA)

    @functools.partial(
        pl.kernel, mesh=_mesh(),
        out_type=[jax.ShapeDtypeStruct((2, NP), jnp.float32)
                  for _ in range(ntab)],
        scratch_types=scratch,
    )
    def prop(src_hbm, dst_hbm, zeros_hbm, *args):
        tabs = args[:ntab]
        outs = args[ntab:2 * ntab]
        src_v, dst_v = args[2 * ntab:2 * ntab + 2]
        msgs = args[2 * ntab + 2:3 * ntab + 2]
        zs = args[3 * ntab + 2:4 * ntab + 2]
        sem = args[4 * ntab + 2]
        sem2 = args[4 * ntab + 3]
        cid = lax.axis_index("c")
        sid = lax.axis_index("s")
        wid = sid * 2 + cid
        for t in range(ntab):
            pltpu.sync_copy(zeros_hbm.at[pl.ds(sid * ZSLICE, ZSLICE)],
                            zs[t].at[pl.ds(sid * ZSLICE, ZSLICE)])
        plsc.subcore_barrier()

        def body(ci, carry):
            rowbase = wid * RPW + ci * K
            pltpu.sync_copy(src_hbm.at[pl.ds(rowbase, K)], src_v)
            pltpu.sync_copy(dst_hbm.at[pl.ds(rowbase, K)], dst_v)
            handles = []
            for t in range(ntab):
                for j in range(K):
                    handles.append(pltpu.async_copy(
                        tabs[t].at[src_v.at[j]], msgs[t].at[j], sem))
            for h in handles:
                h.wait()
            sh = []
            for t in range(ntab):
                for j in range(K):
                    sh.append(pltpu.async_copy(
                        msgs[t].at[j], zs[t].at[dst_v.at[j]], sem2, add=True))
            for h in sh:
                h.wait()
            return carry

        lax.fori_loop(0, NCHUNK, body, 0)
        plsc.subcore_barrier()
        for t in range(ntab):
            pltpu.sync_copy(zs[t].at[pl.ds(sid * ZSLICE, ZSLICE)],
                            outs[t].at[cid, pl.ds(sid * ZSLICE, ZSLICE)])

    return prop


_BR = 56
_GRID = ROWS // _BR


def _rows_spec(nplanes):
    if nplanes == 1:
        return pl.BlockSpec((_BR, LANE), lambda i: (i, 0))
    return pl.BlockSpec((nplanes, _BR, LANE), lambda i: (0, i, 0))


def _smem_spec():
    return pl.BlockSpec(memory_space=pltpu.SMEM)


def _tc_b_body(dega_ref, degb_ref, x0_ref, x1_ref,
               dinv_ref, y10_ref, y11_ref):
    deg = dega_ref[...] + degb_ref[...] + 1.0
    dinv = 1.0 / jnp.sqrt(deg)
    dinv_ref[...] = dinv
    y10_ref[...] = dinv * x0_ref[...]
    y11_ref[...] = dinv * x1_ref[...]


def _tc_d1_body(z1a_ref, z1b_ref, y1_ref, dinv_ref, w1_ref, b1_ref,
                w2_ref, y2_ref):
    dinv = dinv_ref[...]
    q = [dinv * (z1a_ref[g] + z1b_ref[g] + y1_ref[g]) for g in range(2)]
    h1 = []
    for f in range(32):
        acc = q[0] * w1_ref[0, f] + q[1] * w1_ref[1, f] + b1_ref[f]
        h1.append(jnp.maximum(acc, 0.0))
    for g in range(16):
        acc = h1[0] * w2_ref[0, g]
        for f in range(1, 32):
            acc = acc + h1[f] * w2_ref[f, g]
        y2_ref[g] = dinv * acc


def _tc_d2_body(z2a_ref, z2b_ref, y2_ref, dinv_ref, b2_ref, w3_ref,
                y3_ref, hsum_ref):
    i = pl.program_id(0)
    dinv = dinv_ref[...]
    sub = lax.broadcasted_iota(jnp.int32, (_BR, LANE), 0)
    lane = lax.broadcasted_iota(jnp.int32, (_BR, LANE), 1)
    gid = (i * _BR + sub) * LANE + lane
    mask = (gid < N_NODES).astype(jnp.float32)

    @pl.when(i == 0)
    def _init():
        hsum_ref[...] = jnp.zeros((16, 1, LANE), jnp.float32)

    acc3 = None
    for f in range(16):
        h2 = jnp.maximum(
            dinv * (z2a_ref[f] + z2b_ref[f] + y2_ref[f]) + b2_ref[f], 0.0)
        hm = h2 * mask
        hsum_ref[f] = hsum_ref[f] + jnp.sum(hm, axis=0, keepdims=True)
        term = h2 * w3_ref[f, 0]
        acc3 = term if acc3 is None else acc3 + term
    y3_ref[...] = dinv * acc3


def _tc_e_body(z3a_ref, z3b_ref, y3_ref, dinv_ref, b3_ref,
               hsum_ref, wf_ref, bf_ref, choice_ref, value_ref):
    c = dinv_ref[...] * (z3a_ref[...] + z3b_ref[...] + y3_ref[...]) + b3_ref[0]
    sub = lax.broadcasted_iota(jnp.int32, (ROWS, LANE), 0)
    lane = lax.broadcasted_iota(jnp.int32, (ROWS, LANE), 1)
    valid = (sub * LANE + lane) < N_NODES
    cm = jnp.where(valid, c, -jnp.inf)
    m = jnp.max(cm)
    e = jnp.where(valid, jnp.exp(c - m), 0.0)
    s = jnp.sum(e)
    choice_ref[...] = e / s
    val = bf_ref[0]
    for f in range(16):
        val = val + jnp.sum(hsum_ref[f]) / float(N_NODES) * wf_ref[f, 0]
    value_ref[...] = jnp.full((1, LANE), val, jnp.float32)


def _tc_b(dega, degb, x0, x1):
    return pl.pallas_call(
        _tc_b_body,
        grid=(_GRID,),
        in_specs=[_rows_spec(1)] * 4,
        out_specs=[_rows_spec(1)] * 3,
        out_shape=[jax.ShapeDtypeStruct((ROWS, LANE), jnp.float32)] * 3,
    )(dega, degb, x0, x1)


def _tc_d1(z1a, z1b, y1, dinv, W1, b1, W2):
    return pl.pallas_call(
        _tc_d1_body,
        grid=(_GRID,),
        in_specs=[_rows_spec(2), _rows_spec(2), _rows_spec(2), _rows_spec(1),
                  _smem_spec(), _smem_spec(), _smem_spec()],
        out_specs=[_rows_spec(16)],
        out_shape=[jax.ShapeDtypeStruct((16, ROWS, LANE), jnp.float32)],
    )(z1a, z1b, y1, dinv, W1, b1, W2)[0]


def _tc_d2(z2a, z2b, y2, dinv, b2, W3):
    return pl.pallas_call(
        _tc_d2_body,
        grid=(_GRID,),
        in_specs=[_rows_spec(16), _rows_spec(16), _rows_spec(16),
                  _rows_spec(1), _smem_spec(), _smem_spec()],
        out_specs=[_rows_spec(1),
                   pl.BlockSpec((16, 1, LANE), lambda i: (0, 0, 0))],
        out_shape=[jax.ShapeDtypeStruct((ROWS, LANE), jnp.float32),
                   jax.ShapeDtypeStruct((16, 1, LANE), jnp.float32)],
    )(z2a, z2b, y2, dinv, b2, W3)


def _tc_e(z3a, z3b, y3, dinv, b3, hsum, Wf, bf):
    return pl.pallas_call(
        _tc_e_body,
        grid=(),
        in_specs=[pl.BlockSpec((ROWS, LANE), lambda: (0, 0))] * 4
        + [_smem_spec(), pl.BlockSpec((16, 1, LANE), lambda: (0, 0, 0)),
           _smem_spec(), _smem_spec()],
        out_specs=[pl.BlockSpec((ROWS, LANE), lambda: (0, 0)),
                   pl.BlockSpec((1, LANE), lambda: (0, 0))],
        out_shape=[jax.ShapeDtypeStruct((ROWS, LANE), jnp.float32),
                   jax.ShapeDtypeStruct((1, LANE), jnp.float32)],
    )(z3a, z3b, y3, dinv, b3, hsum, Wf, bf)


@jax.jit
def _run(x, edge_index, W1, b1, W2, b2, W3, b3, Wf, bf):
    f32 = jnp.float32
    pad_e = EP - E_EDGES
    ar = jnp.arange(pad_e, dtype=jnp.int32)
    src_rows = jnp.concatenate(
        [edge_index[0], ar % N_NODES]).reshape(ROWS_E, LANE)
    dst_rows = jnp.concatenate(
        [edge_index[1], N_NODES + ar % (NP - N_NODES)]).reshape(ROWS_E, LANE)
    zeros1 = jnp.zeros((NP,), f32)

    xp = jnp.pad(x, ((0, NP - N_NODES), (0, 0)))
    x0 = xp[:, 0].reshape(ROWS, LANE)
    x1 = xp[:, 1].reshape(ROWS, LANE)

    deg = _get_sc_degree()(dst_rows, zeros1)
    dega = deg[0].reshape(ROWS, LANE)
    degb = deg[1].reshape(ROWS, LANE)
    dinv, y10, y11 = _tc_b(dega, degb, x0, x1)

    z1 = _make_prop_cols(2)(src_rows, dst_rows, zeros1,
                            y10.reshape(NP), y11.reshape(NP))
    z1a = jnp.stack([z1[0][0], z1[1][0]]).reshape(2, ROWS, LANE)
    z1b = jnp.stack([z1[0][1], z1[1][1]]).reshape(2, ROWS, LANE)
    y1 = jnp.stack([y10, y11])
    y2 = _tc_d1(z1a, z1b, y1, dinv, W1, b1, W2)

    z2 = _make_prop_cols(16)(src_rows, dst_rows, zeros1,
                             *[y2[f].reshape(NP) for f in range(16)])
    z2p = jnp.stack([z2[f] for f in range(16)], axis=1)
    z2p = z2p.reshape(2, 16, ROWS, LANE)
    y3, hsum = _tc_d2(z2p[0], z2p[1], y2, dinv, b2, W3)

    z3 = _make_prop_cols(1)(src_rows, dst_rows, zeros1,
                            y3.reshape(NP))[0]
    choice2d, value2d = _tc_e(
        z3[0].reshape(ROWS, LANE), z3[1].reshape(ROWS, LANE),
        y3, dinv, b3, hsum, Wf, bf)
    choice = choice2d.reshape(NP, 1)[:N_NODES]
    value = value2d[:1, :1]
    return choice, value


def kernel(x, edge_index, choices, W1, b1, W2, b2, W3, b3, Wf, bf):
    return _run(x, edge_index, W1, b1, W2, b2, W3, b3, Wf, bf)


# speedup vs baseline: 31.2452x; 1.0001x over previous
import functools

import jax
import jax.numpy as jnp
from jax import lax
from jax.experimental import pallas as pl
from jax.experimental.pallas import tpu as pltpu
from jax.experimental.pallas import tpu_sc as plsc

N_NODES = 100000
E_EDGES = 6400000
LANE = 128
ROWS = 784
NP = ROWS * LANE
NW = 32
K = 8
RPW = 1568
EP = NW * RPW * LANE
ROWS_E = EP // LANE
NCHUNK = RPW // K
ZSLICE = NP // 16

@functools.cache
def _mesh():
    return plsc.VectorSubcoreMesh(core_axis_name="c", subcore_axis_name="s")


@functools.cache
def _get_sc_degree():
    return pl.kernel(
        _sc_degree_body, mesh=_mesh(),
        out_type=jax.ShapeDtypeStruct((2, NP), jnp.float32),
        scratch_types=[
            pltpu.VMEM((K * LANE,), jnp.int32),
            pltpu.VMEM((K * LANE,), jnp.float32),
            pltpu.VMEM_SHARED((NP,), jnp.float32),
            pltpu.SemaphoreType.DMA,
        ],
    )


def _sc_degree_body(dst_hbm, zeros_hbm, out_hbm, dst_v, ones_v, z_sh, sem):
    cid = lax.axis_index("c")
    sid = lax.axis_index("s")
    wid = sid * 2 + cid
    pltpu.sync_copy(zeros_hbm.at[pl.ds(sid * ZSLICE, ZSLICE)],
                    z_sh.at[pl.ds(sid * ZSLICE, ZSLICE)])
    for t in range(K * LANE // 16):
        ones_v[pl.ds(t * 16, 16)] = jnp.ones((16,), jnp.float32)
    plsc.subcore_barrier()

    def body(ci, carry):
        base = (wid * RPW + ci * K) * LANE
        pltpu.sync_copy(dst_hbm.at[pl.ds(base, K * LANE)], dst_v)
        pltpu.async_copy(ones_v, z_sh.at[dst_v], sem, add=True).wait()
        return carry

    lax.fori_loop(0, NCHUNK, body, 0)
    plsc.subcore_barrier()
    pltpu.sync_copy(z_sh.at[pl.ds(sid * ZSLICE, ZSLICE)],
                    out_hbm.at[cid, pl.ds(sid * ZSLICE, ZSLICE)])


@functools.cache
def _make_prop_cols(ntab):
    scratch = [
        pltpu.VMEM((K * LANE,), jnp.int32),
        pltpu.VMEM((K * LANE,), jnp.int32),
    ]
    scratch += [pltpu.VMEM((K * LANE,), jnp.float32) for _ in range(ntab)]
    scratch += [pltpu.VMEM_SHARED((NP,), jnp.float32) for _ in range(ntab)]
    scratch.append(pltpu.SemaphoreType.DMA)
    scratch.append(pltpu.SemaphoreType.DMA)

    @functools.partial(
        pl.kernel, mesh=_mesh(),
        out_type=[jax.ShapeDtypeStruct((2, NP), jnp.float32)
                  for _ in range(ntab)],
        scratch_types=scratch,
    )
    def prop(src_hbm, dst_hbm, zeros_hbm, *args):
        tabs = args[:ntab]
        outs = args[ntab:2 * ntab]
        src_v, dst_v = args[2 * ntab:2 * ntab + 2]
        msgs = args[2 * ntab + 2:3 * ntab + 2]
        zs = args[3 * ntab + 2:4 * ntab + 2]
        sem = args[4 * ntab + 2]
        sem2 = args[4 * ntab + 3]
        cid = lax.axis_index("c")
        sid = lax.axis_index("s")
        wid = sid * 2 + cid
        for t in range(ntab):
            pltpu.sync_copy(zeros_hbm.at[pl.ds(sid * ZSLICE, ZSLICE)],
                            zs[t].at[pl.ds(sid * ZSLICE, ZSLICE)])
        plsc.subcore_barrier()

        def body(ci, carry):
            base = (wid * RPW + ci * K) * LANE
            pltpu.sync_copy(src_hbm.at[pl.ds(base, K * LANE)], src_v)
            pltpu.sync_copy(dst_hbm.at[pl.ds(base, K * LANE)], dst_v)
            handles = [pltpu.async_copy(tabs[t].at[src_v], msgs[t], sem)
                       for t in range(ntab)]
            for h in handles:
                h.wait()
            sh = [pltpu.async_copy(msgs[t], zs[t].at[dst_v], sem2, add=True)
                  for t in range(ntab)]
            for h in sh:
                h.wait()
            return carry

        lax.fori_loop(0, NCHUNK, body, 0)
        plsc.subcore_barrier()
        for t in range(ntab):
            pltpu.sync_copy(zs[t].at[pl.ds(sid * ZSLICE, ZSLICE)],
                            outs[t].at[cid, pl.ds(sid * ZSLICE, ZSLICE)])

    return prop


_BR = 56
_GRID = ROWS // _BR


def _rows_spec(nplanes):
    if nplanes == 1:
        return pl.BlockSpec((_BR, LANE), lambda i: (i, 0))
    return pl.BlockSpec((nplanes, _BR, LANE), lambda i: (0, i, 0))


def _smem_spec():
    return pl.BlockSpec(memory_space=pltpu.SMEM)


def _tc_b_body(dega_ref, degb_ref, x0_ref, x1_ref,
               dinv_ref, y10_ref, y11_ref):
    deg = dega_ref[...] + degb_ref[...] + 1.0
    dinv = 1.0 / jnp.sqrt(deg)
    dinv_ref[...] = dinv
    y10_ref[...] = dinv * x0_ref[...]
    y11_ref[...] = dinv * x1_ref[...]


def _tc_d1_body(z1a_ref, z1b_ref, y1_ref, dinv_ref, w1_ref, b1_ref,
                w2_ref, y2_ref):
    dinv = dinv_ref[...]
    q = [dinv * (z1a_ref[g] + z1b_ref[g] + y1_ref[g]) for g in range(2)]
    h1 = []
    for f in range(32):
        acc = q[0] * w1_ref[0, f] + q[1] * w1_ref[1, f] + b1_ref[f]
        h1.append(jnp.maximum(acc, 0.0))
    for g in range(16):
        acc = h1[0] * w2_ref[0, g]
        for f in range(1, 32):
            acc = acc + h1[f] * w2_ref[f, g]
        y2_ref[g] = dinv * acc


def _tc_d2_body(z2a_ref, z2b_ref, y2_ref, dinv_ref, b2_ref, w3_ref,
                y3_ref, hsum_ref):
    i = pl.program_id(0)
    dinv = dinv_ref[...]
    sub = lax.broadcasted_iota(jnp.int32, (_BR, LANE), 0)
    lane = lax.broadcasted_iota(jnp.int32, (_BR, LANE), 1)
    gid = (i * _BR + sub) * LANE + lane
    mask = (gid < N_NODES).astype(jnp.float32)

    @pl.when(i == 0)
    def _init():
        hsum_ref[...] = jnp.zeros((16, 1, LANE), jnp.float32)

    acc3 = None
    for f in range(16):
        h2 = jnp.maximum(
            dinv * (z2a_ref[f] + z2b_ref[f] + y2_ref[f]) + b2_ref[f], 0.0)
        hm = h2 * mask
        hsum_ref[f] = hsum_ref[f] + jnp.sum(hm, axis=0, keepdims=True)
        term = h2 * w3_ref[f, 0]
        acc3 = term if acc3 is None else acc3 + term
    y3_ref[...] = dinv * acc3


def _tc_e_body(z3a_ref, z3b_ref, y3_ref, dinv_ref, b3_ref,
               hsum_ref, wf_ref, bf_ref, choice_ref, value_ref):
    c = dinv_ref[...] * (z3a_ref[...] + z3b_ref[...] + y3_ref[...]) + b3_ref[0]
    sub = lax.broadcasted_iota(jnp.int32, (ROWS, LANE), 0)
    lane = lax.broadcasted_iota(jnp.int32, (ROWS, LANE), 1)
    valid = (sub * LANE + lane) < N_NODES
    cm = jnp.where(valid, c, -jnp.inf)
    m = jnp.max(cm)
    e = jnp.where(valid, jnp.exp(c - m), 0.0)
    s = jnp.sum(e)
    choice_ref[...] = e / s
    val = bf_ref[0]
    for f in range(16):
        val = val + jnp.sum(hsum_ref[f]) / float(N_NODES) * wf_ref[f, 0]
    value_ref[...] = jnp.full((1, LANE), val, jnp.float32)


def _tc_b(dega, degb, x0, x1):
    return pl.pallas_call(
        _tc_b_body,
        grid=(_GRID,),
        in_specs=[_rows_spec(1)] * 4,
        out_specs=[_rows_spec(1)] * 3,
        out_shape=[jax.ShapeDtypeStruct((ROWS, LANE), jnp.float32)] * 3,
    )(dega, degb, x0, x1)


def _tc_d1(z1a, z1b, y1, dinv, W1, b1, W2):
    return pl.pallas_call(
        _tc_d1_body,
        grid=(_GRID,),
        in_specs=[_rows_spec(2), _rows_spec(2), _rows_spec(2), _rows_spec(1),
                  _smem_spec(), _smem_spec(), _smem_spec()],
        out_specs=[_rows_spec(16)],
        out_shape=[jax.ShapeDtypeStruct((16, ROWS, LANE), jnp.float32)],
    )(z1a, z1b, y1, dinv, W1, b1, W2)[0]


def _tc_d2(z2a, z2b, y2, dinv, b2, W3):
    return pl.pallas_call(
        _tc_d2_body,
        grid=(_GRID,),
        in_specs=[_rows_spec(16), _rows_spec(16), _rows_spec(16),
                  _rows_spec(1), _smem_spec(), _smem_spec()],
        out_specs=[_rows_spec(1),
                   pl.BlockSpec((16, 1, LANE), lambda i: (0, 0, 0))],
        out_shape=[jax.ShapeDtypeStruct((ROWS, LANE), jnp.float32),
                   jax.ShapeDtypeStruct((16, 1, LANE), jnp.float32)],
    )(z2a, z2b, y2, dinv, b2, W3)


def _tc_e(z3a, z3b, y3, dinv, b3, hsum, Wf, bf):
    return pl.pallas_call(
        _tc_e_body,
        grid=(),
        in_specs=[pl.BlockSpec((ROWS, LANE), lambda: (0, 0))] * 4
        + [_smem_spec(), pl.BlockSpec((16, 1, LANE), lambda: (0, 0, 0)),
           _smem_spec(), _smem_spec()],
        out_specs=[pl.BlockSpec((ROWS, LANE), lambda: (0, 0)),
                   pl.BlockSpec((1, LANE), lambda: (0, 0))],
        out_shape=[jax.ShapeDtypeStruct((ROWS, LANE), jnp.float32),
                   jax.ShapeDtypeStruct((1, LANE), jnp.float32)],
    )(z3a, z3b, y3, dinv, b3, hsum, Wf, bf)


@jax.jit
def _run(x, edge_index, W1, b1, W2, b2, W3, b3, Wf, bf):
    f32 = jnp.float32
    pad_e = EP - E_EDGES
    ar = jnp.arange(pad_e, dtype=jnp.int32)
    src_rows = jnp.concatenate([edge_index[0], ar % N_NODES])
    dst_rows = jnp.concatenate(
        [edge_index[1], N_NODES + ar % (NP - N_NODES)])
    zeros1 = jnp.zeros((NP,), f32)

    xp = jnp.pad(x, ((0, NP - N_NODES), (0, 0)))
    x0 = xp[:, 0].reshape(ROWS, LANE)
    x1 = xp[:, 1].reshape(ROWS, LANE)

    deg = _get_sc_degree()(dst_rows, zeros1)
    dega = deg[0].reshape(ROWS, LANE)
    degb = deg[1].reshape(ROWS, LANE)
    dinv, y10, y11 = _tc_b(dega, degb, x0, x1)

    z1 = _make_prop_cols(2)(src_rows, dst_rows, zeros1,
                            y10.reshape(NP), y11.reshape(NP))
    z1a = jnp.stack([z1[0][0], z1[1][0]]).reshape(2, ROWS, LANE)
    z1b = jnp.stack([z1[0][1], z1[1][1]]).reshape(2, ROWS, LANE)
    y1 = jnp.stack([y10, y11])
    y2 = _tc_d1(z1a, z1b, y1, dinv, W1, b1, W2)

    z2 = _make_prop_cols(16)(src_rows, dst_rows, zeros1,
                             *[y2[f].reshape(NP) for f in range(16)])
    z2p = jnp.stack([z2[f] for f in range(16)], axis=1)
    z2p = z2p.reshape(2, 16, ROWS, LANE)
    y3, hsum = _tc_d2(z2p[0], z2p[1], y2, dinv, b2, W3)

    z3 = _make_prop_cols(1)(src_rows, dst_rows, zeros1,
                            y3.reshape(NP))[0]
    choice2d, value2d = _tc_e(
        z3[0].reshape(ROWS, LANE), z3[1].reshape(ROWS, LANE),
        y3, dinv, b3, hsum, Wf, bf)
    choice = choice2d.reshape(NP, 1)[:N_NODES]
    value = value2d[:1, :1]
    return choice, value


def kernel(x, edge_index, choices, W1, b1, W2, b2, W3, b3, Wf, bf):
    return _run(x, edge_index, W1, b1, W2, b2, W3, b3, Wf, bf)


# speedup vs baseline: 35.9334x; 1.1500x over previous
import functools

import jax
import jax.numpy as jnp
from jax import lax
from jax.experimental import pallas as pl
from jax.experimental.pallas import tpu as pltpu
from jax.experimental.pallas import tpu_sc as plsc

N_NODES = 100000
E_EDGES = 6400000
LANE = 128
ROWS = 784
NP = ROWS * LANE
NW = 32
K = 8
RPW = 1568
EP = NW * RPW * LANE
ROWS_E = EP // LANE
NCHUNK = RPW // K
ZSLICE = NP // 16

@functools.cache
def _mesh():
    return plsc.VectorSubcoreMesh(core_axis_name="c", subcore_axis_name="s")


@functools.cache
def _get_sc_degree():
    return pl.kernel(
        _sc_degree_body, mesh=_mesh(),
        out_type=jax.ShapeDtypeStruct((2, NP), jnp.float32),
        scratch_types=[
            pltpu.VMEM((K * LANE,), jnp.int32),
            pltpu.VMEM((K * LANE,), jnp.float32),
            pltpu.VMEM_SHARED((NP,), jnp.float32),
            pltpu.SemaphoreType.DMA,
        ],
    )


def _sc_degree_body(dst_hbm, zeros_hbm, out_hbm, dst_v, ones_v, z_sh, sem):
    cid = lax.axis_index("c")
    sid = lax.axis_index("s")
    wid = sid * 2 + cid
    pltpu.sync_copy(zeros_hbm.at[pl.ds(sid * ZSLICE, ZSLICE)],
                    z_sh.at[pl.ds(sid * ZSLICE, ZSLICE)])
    for t in range(K * LANE // 16):
        ones_v[pl.ds(t * 16, 16)] = jnp.ones((16,), jnp.float32)
    plsc.subcore_barrier()

    def body(ci, carry):
        base = (wid * RPW + ci * K) * LANE
        pltpu.sync_copy(dst_hbm.at[pl.ds(base, K * LANE)], dst_v)
        pltpu.async_copy(ones_v, z_sh.at[dst_v], sem, add=True).wait()
        return carry

    lax.fori_loop(0, NCHUNK, body, 0)
    plsc.subcore_barrier()
    pltpu.sync_copy(z_sh.at[pl.ds(sid * ZSLICE, ZSLICE)],
                    out_hbm.at[cid, pl.ds(sid * ZSLICE, ZSLICE)])


@functools.cache
def _make_prop_cols(ntab):
    k = K // 2 if ntab > 8 else K
    nchunk = RPW // k
    KL = k * LANE
    scratch = [pltpu.VMEM((KL,), jnp.int32) for _ in range(4)]
    scratch += [pltpu.VMEM((KL,), jnp.float32) for _ in range(2 * ntab)]
    scratch += [pltpu.VMEM_SHARED((NP,), jnp.float32) for _ in range(ntab)]
    scratch.append(pltpu.SemaphoreType.DMA)
    scratch.append(pltpu.SemaphoreType.DMA)

    @functools.partial(
        pl.kernel, mesh=_mesh(),
        out_type=[jax.ShapeDtypeStruct((2, NP), jnp.float32)
                  for _ in range(ntab)],
        scratch_types=scratch,
    )
    def prop(src_hbm, dst_hbm, zeros_hbm, *args):
        tabs = args[:ntab]
        outs = args[ntab:2 * ntab]
        a = 2 * ntab
        src_v = args[a:a + 2]
        dst_v = args[a + 2:a + 4]
        msgs = args[a + 4:a + 4 + 2 * ntab]
        zs = args[a + 4 + 2 * ntab:a + 4 + 3 * ntab]
        sem = args[a + 4 + 3 * ntab]
        sem2 = args[a + 5 + 3 * ntab]
        cid = lax.axis_index("c")
        sid = lax.axis_index("s")
        wid = sid * 2 + cid
        for t in range(ntab):
            pltpu.sync_copy(zeros_hbm.at[pl.ds(sid * ZSLICE, ZSLICE)],
                            zs[t].at[pl.ds(sid * ZSLICE, ZSLICE)])
        plsc.subcore_barrier()
        base0 = wid * RPW * LANE

        def load_idx(base, b):
            pltpu.sync_copy(src_hbm.at[pl.ds(base, KL)], src_v[b])
            pltpu.sync_copy(dst_hbm.at[pl.ds(base, KL)], dst_v[b])

        def fire_g(b):
            for t in range(ntab):
                pltpu.async_copy(tabs[t].at[src_v[b]], msgs[2 * t + b], sem)

        def drain_g(b):
            for t in range(ntab):
                pltpu.make_async_copy(
                    tabs[t].at[src_v[b]], msgs[2 * t + b], sem).wait()

        def fire_s(b):
            for t in range(ntab):
                pltpu.async_copy(msgs[2 * t + b], zs[t].at[dst_v[b]],
                                 sem2, add=True)

        def drain_s(b):
            for t in range(ntab):
                pltpu.make_async_copy(msgs[2 * t + b], zs[t].at[dst_v[b]],
                                      sem2).wait()

        load_idx(base0, 0)
        fire_g(0)
        load_idx(base0 + KL, 1)
        fire_g(1)
        drain_g(0)
        fire_s(0)

        def body(i, carry):
            for b in (0, 1):
                base = base0 + (2 * i + b) * KL
                drain_s(b)
                load_idx(base, b)
                fire_g(b)
                drain_g(1 - b)
                fire_s(1 - b)
            return carry

        lax.fori_loop(1, nchunk // 2, body, 0)
        drain_g(1)
        fire_s(1)
        drain_s(0)
        drain_s(1)
        plsc.subcore_barrier()
        for t in range(ntab):
            pltpu.sync_copy(zs[t].at[pl.ds(sid * ZSLICE, ZSLICE)],
                            outs[t].at[cid, pl.ds(sid * ZSLICE, ZSLICE)])

    return prop


_BR = 56
_GRID = ROWS // _BR


def _rows_spec(nplanes):
    if nplanes == 1:
        return pl.BlockSpec((_BR, LANE), lambda i: (i, 0))
    return pl.BlockSpec((nplanes, _BR, LANE), lambda i: (0, i, 0))


def _smem_spec():
    return pl.BlockSpec(memory_space=pltpu.SMEM)


def _tc_b_body(dega_ref, degb_ref, x0_ref, x1_ref,
               dinv_ref, y10_ref, y11_ref):
    deg = dega_ref[...] + degb_ref[...] + 1.0
    dinv = 1.0 / jnp.sqrt(deg)
    dinv_ref[...] = dinv
    y10_ref[...] = dinv * x0_ref[...]
    y11_ref[...] = dinv * x1_ref[...]


def _tc_d1_body(z1a_ref, z1b_ref, y1_ref, dinv_ref, w1_ref, b1_ref,
                w2_ref, y2_ref):
    dinv = dinv_ref[...]
    q = [dinv * (z1a_ref[g] + z1b_ref[g] + y1_ref[g]) for g in range(2)]
    h1 = []
    for f in range(32):
        acc = q[0] * w1_ref[0, f] + q[1] * w1_ref[1, f] + b1_ref[f]
        h1.append(jnp.maximum(acc, 0.0))
    for g in range(16):
        acc = h1[0] * w2_ref[0, g]
        for f in range(1, 32):
            acc = acc + h1[f] * w2_ref[f, g]
        y2_ref[g] = dinv * acc


def _tc_d2_body(z2a_ref, z2b_ref, y2_ref, dinv_ref, b2_ref, w3_ref,
                y3_ref, hsum_ref):
    i = pl.program_id(0)
    dinv = dinv_ref[...]
    sub = lax.broadcasted_iota(jnp.int32, (_BR, LANE), 0)
    lane = lax.broadcasted_iota(jnp.int32, (_BR, LANE), 1)
    gid = (i * _BR + sub) * LANE + lane
    mask = (gid < N_NODES).astype(jnp.float32)

    @pl.when(i == 0)
    def _init():
        hsum_ref[...] = jnp.zeros((16, 1, LANE), jnp.float32)

    acc3 = None
    for f in range(16):
        h2 = jnp.maximum(
            dinv * (z2a_ref[f] + z2b_ref[f] + y2_ref[f]) + b2_ref[f], 0.0)
        hm = h2 * mask
        hsum_ref[f] = hsum_ref[f] + jnp.sum(hm, axis=0, keepdims=True)
        term = h2 * w3_ref[f, 0]
        acc3 = term if acc3 is None else acc3 + term
    y3_ref[...] = dinv * acc3


def _tc_e_body(z3a_ref, z3b_ref, y3_ref, dinv_ref, b3_ref,
               hsum_ref, wf_ref, bf_ref, choice_ref, value_ref):
    c = dinv_ref[...] * (z3a_ref[...] + z3b_ref[...] + y3_ref[...]) + b3_ref[0]
    sub = lax.broadcasted_iota(jnp.int32, (ROWS, LANE), 0)
    lane = lax.broadcasted_iota(jnp.int32, (ROWS, LANE), 1)
    valid = (sub * LANE + lane) < N_NODES
    cm = jnp.where(valid, c, -jnp.inf)
    m = jnp.max(cm)
    e = jnp.where(valid, jnp.exp(c - m), 0.0)
    s = jnp.sum(e)
    choice_ref[...] = e / s
    val = bf_ref[0]
    for f in range(16):
        val = val + jnp.sum(hsum_ref[f]) / float(N_NODES) * wf_ref[f, 0]
    value_ref[...] = jnp.full((1, LANE), val, jnp.float32)


def _tc_b(dega, degb, x0, x1):
    return pl.pallas_call(
        _tc_b_body,
        grid=(_GRID,),
        in_specs=[_rows_spec(1)] * 4,
        out_specs=[_rows_spec(1)] * 3,
        out_shape=[jax.ShapeDtypeStruct((ROWS, LANE), jnp.float32)] * 3,
    )(dega, degb, x0, x1)


def _tc_d1(z1a, z1b, y1, dinv, W1, b1, W2):
    return pl.pallas_call(
        _tc_d1_body,
        grid=(_GRID,),
        in_specs=[_rows_spec(2), _rows_spec(2), _rows_spec(2), _rows_spec(1),
                  _smem_spec(), _smem_spec(), _smem_spec()],
        out_specs=[_rows_spec(16)],
        out_shape=[jax.ShapeDtypeStruct((16, ROWS, LANE), jnp.float32)],
    )(z1a, z1b, y1, dinv, W1, b1, W2)[0]


def _tc_d2(z2a, z2b, y2, dinv, b2, W3):
    return pl.pallas_call(
        _tc_d2_body,
        grid=(_GRID,),
        in_specs=[_rows_spec(16), _rows_spec(16), _rows_spec(16),
                  _rows_spec(1), _smem_spec(), _smem_spec()],
        out_specs=[_rows_spec(1),
                   pl.BlockSpec((16, 1, LANE), lambda i: (0, 0, 0))],
        out_shape=[jax.ShapeDtypeStruct((ROWS, LANE), jnp.float32),
                   jax.ShapeDtypeStruct((16, 1, LANE), jnp.float32)],
    )(z2a, z2b, y2, dinv, b2, W3)


def _tc_e(z3a, z3b, y3, dinv, b3, hsum, Wf, bf):
    return pl.pallas_call(
        _tc_e_body,
        grid=(),
        in_specs=[pl.BlockSpec((ROWS, LANE), lambda: (0, 0))] * 4
        + [_smem_spec(), pl.BlockSpec((16, 1, LANE), lambda: (0, 0, 0)),
           _smem_spec(), _smem_spec()],
        out_specs=[pl.BlockSpec((ROWS, LANE), lambda: (0, 0)),
                   pl.BlockSpec((1, LANE), lambda: (0, 0))],
        out_shape=[jax.ShapeDtypeStruct((ROWS, LANE), jnp.float32),
                   jax.ShapeDtypeStruct((1, LANE), jnp.float32)],
    )(z3a, z3b, y3, dinv, b3, hsum, Wf, bf)


@jax.jit
def _run(x, edge_index, W1, b1, W2, b2, W3, b3, Wf, bf):
    f32 = jnp.float32
    pad_e = EP - E_EDGES
    ar = jnp.arange(pad_e, dtype=jnp.int32)
    src_rows = jnp.concatenate([edge_index[0], ar % N_NODES])
    dst_rows = jnp.concatenate(
        [edge_index[1], N_NODES + ar % (NP - N_NODES)])
    zeros1 = jnp.zeros((NP,), f32)

    xp = jnp.pad(x, ((0, NP - N_NODES), (0, 0)))
    x0 = xp[:, 0].reshape(ROWS, LANE)
    x1 = xp[:, 1].reshape(ROWS, LANE)

    deg = _get_sc_degree()(dst_rows, zeros1)
    dega = deg[0].reshape(ROWS, LANE)
    degb = deg[1].reshape(ROWS, LANE)
    dinv, y10, y11 = _tc_b(dega, degb, x0, x1)

    z1 = _make_prop_cols(2)(src_rows, dst_rows, zeros1,
                            y10.reshape(NP), y11.reshape(NP))
    z1a = jnp.stack([z1[0][0], z1[1][0]]).reshape(2, ROWS, LANE)
    z1b = jnp.stack([z1[0][1], z1[1][1]]).reshape(2, ROWS, LANE)
    y1 = jnp.stack([y10, y11])
    y2 = _tc_d1(z1a, z1b, y1, dinv, W1, b1, W2)

    z2 = _make_prop_cols(16)(src_rows, dst_rows, zeros1,
                             *[y2[f].reshape(NP) for f in range(16)])
    z2p = jnp.stack([z2[f] for f in range(16)], axis=1)
    z2p = z2p.reshape(2, 16, ROWS, LANE)
    y3, hsum = _tc_d2(z2p[0], z2p[1], y2, dinv, b2, W3)

    z3 = _make_prop_cols(1)(src_rows, dst_rows, zeros1,
                            y3.reshape(NP))[0]
    choice2d, value2d = _tc_e(
        z3[0].reshape(ROWS, LANE), z3[1].reshape(ROWS, LANE),
        y3, dinv, b3, hsum, Wf, bf)
    choice = choice2d.reshape(NP, 1)[:N_NODES]
    value = value2d[:1, :1]
    return choice, value


def kernel(x, edge_index, choices, W1, b1, W2, b2, W3, b3, Wf, bf):
    return _run(x, edge_index, W1, b1, W2, b2, W3, b3, Wf, bf)


# speedup vs baseline: 36.5372x; 1.0168x over previous
import functools

import jax
import jax.numpy as jnp
from jax import lax
from jax.experimental import pallas as pl
from jax.experimental.pallas import tpu as pltpu
from jax.experimental.pallas import tpu_sc as plsc

N_NODES = 100000
E_EDGES = 6400000
LANE = 128
ROWS = 784
NP = ROWS * LANE
NW = 32
K = 8
RPW = 1568
EP = NW * RPW * LANE
ROWS_E = EP // LANE
NCHUNK = RPW // K
ZSLICE = NP // 16

@functools.cache
def _mesh():
    return plsc.VectorSubcoreMesh(core_axis_name="c", subcore_axis_name="s")


@functools.cache
def _get_sc_degree():
    return pl.kernel(
        _sc_degree_body, mesh=_mesh(),
        out_type=jax.ShapeDtypeStruct((2, NP), jnp.float32),
        scratch_types=[
            pltpu.VMEM((K * LANE,), jnp.int32),
            pltpu.VMEM((K * LANE,), jnp.float32),
            pltpu.VMEM_SHARED((NP,), jnp.float32),
            pltpu.SemaphoreType.DMA,
        ],
    )


def _sc_degree_body(dst_hbm, zeros_hbm, out_hbm, dst_v, ones_v, z_sh, sem):
    cid = lax.axis_index("c")
    sid = lax.axis_index("s")
    wid = sid * 2 + cid
    pltpu.sync_copy(zeros_hbm.at[pl.ds(sid * ZSLICE, ZSLICE)],
                    z_sh.at[pl.ds(sid * ZSLICE, ZSLICE)])
    for t in range(K * LANE // 16):
        ones_v[pl.ds(t * 16, 16)] = jnp.ones((16,), jnp.float32)
    plsc.subcore_barrier()

    def body(ci, carry):
        base = (wid * RPW + ci * K) * LANE
        pltpu.sync_copy(dst_hbm.at[pl.ds(base, K * LANE)], dst_v)
        pltpu.async_copy(ones_v, z_sh.at[dst_v], sem, add=True).wait()
        return carry

    lax.fori_loop(0, NCHUNK, body, 0)
    plsc.subcore_barrier()
    pltpu.sync_copy(z_sh.at[pl.ds(sid * ZSLICE, ZSLICE)],
                    out_hbm.at[cid, pl.ds(sid * ZSLICE, ZSLICE)])


@functools.cache
def _make_prop_cols(ntab):
    KL = K * LANE
    scratch = [pltpu.VMEM((KL,), jnp.int32) for _ in range(4)]
    scratch += [pltpu.VMEM((KL,), jnp.float32) for _ in range(ntab)]
    scratch += [pltpu.VMEM_SHARED((NP,), jnp.float32) for _ in range(ntab)]
    scratch.append(pltpu.SemaphoreType.DMA)
    scratch.append(pltpu.SemaphoreType.DMA)

    @functools.partial(
        pl.kernel, mesh=_mesh(),
        out_type=[jax.ShapeDtypeStruct((2, NP), jnp.float32)
                  for _ in range(ntab)],
        scratch_types=scratch,
    )
    def prop(src_hbm, dst_hbm, zeros_hbm, *args):
        tabs = args[:ntab]
        outs = args[ntab:2 * ntab]
        a = 2 * ntab
        src_v = args[a:a + 2]
        dst_v = args[a + 2:a + 4]
        msgs = args[a + 4:a + 4 + ntab]
        zs = args[a + 4 + ntab:a + 4 + 2 * ntab]
        sem = args[a + 4 + 2 * ntab]
        sem2 = args[a + 5 + 2 * ntab]
        cid = lax.axis_index("c")
        sid = lax.axis_index("s")
        wid = sid * 2 + cid
        for t in range(ntab):
            pltpu.sync_copy(zeros_hbm.at[pl.ds(sid * ZSLICE, ZSLICE)],
                            zs[t].at[pl.ds(sid * ZSLICE, ZSLICE)])
        plsc.subcore_barrier()
        base0 = wid * RPW * LANE

        def load_idx(base, b):
            pltpu.sync_copy(src_hbm.at[pl.ds(base, KL)], src_v[b])
            pltpu.sync_copy(dst_hbm.at[pl.ds(base, KL)], dst_v[b])

        def do_chunk(base, b, first):
            load_idx(base, b)
            for t in range(ntab):
                if not first:
                    pltpu.make_async_copy(
                        msgs[t], zs[t].at[dst_v[1 - b]], sem2).wait()
                pltpu.async_copy(tabs[t].at[src_v[b]], msgs[t], sem)
            for t in range(ntab):
                pltpu.make_async_copy(
                    tabs[t].at[src_v[b]], msgs[t], sem).wait()
                pltpu.async_copy(msgs[t], zs[t].at[dst_v[b]], sem2,
                                 add=True)

        do_chunk(base0, 0, True)
        do_chunk(base0 + KL, 1, False)

        def body(i, carry):
            for b in (0, 1):
                do_chunk(base0 + (2 * i + b) * KL, b, False)
            return carry

        lax.fori_loop(1, NCHUNK // 2, body, 0)
        for t in range(ntab):
            pltpu.make_async_copy(msgs[t], zs[t].at[dst_v[1]], sem2).wait()
        plsc.subcore_barrier()
        for t in range(ntab):
            pltpu.sync_copy(zs[t].at[pl.ds(sid * ZSLICE, ZSLICE)],
                            outs[t].at[cid, pl.ds(sid * ZSLICE, ZSLICE)])

    return prop


_BR = 56
_GRID = ROWS // _BR


def _rows_spec(nplanes):
    if nplanes == 1:
        return pl.BlockSpec((_BR, LANE), lambda i: (i, 0))
    return pl.BlockSpec((nplanes, _BR, LANE), lambda i: (0, i, 0))


def _smem_spec():
    return pl.BlockSpec(memory_space=pltpu.SMEM)


def _tc_b_body(dega_ref, degb_ref, x0_ref, x1_ref,
               dinv_ref, y10_ref, y11_ref):
    deg = dega_ref[...] + degb_ref[...] + 1.0
    dinv = 1.0 / jnp.sqrt(deg)
    dinv_ref[...] = dinv
    y10_ref[...] = dinv * x0_ref[...]
    y11_ref[...] = dinv * x1_ref[...]


def _tc_d1_body(z1a_ref, z1b_ref, y1_ref, dinv_ref, w1_ref, b1_ref,
                w2_ref, y2_ref):
    dinv = dinv_ref[...]
    q = [dinv * (z1a_ref[g] + z1b_ref[g] + y1_ref[g]) for g in range(2)]
    h1 = []
    for f in range(32):
        acc = q[0] * w1_ref[0, f] + q[1] * w1_ref[1, f] + b1_ref[f]
        h1.append(jnp.maximum(acc, 0.0))
    for g in range(16):
        acc = h1[0] * w2_ref[0, g]
        for f in range(1, 32):
            acc = acc + h1[f] * w2_ref[f, g]
        y2_ref[g] = dinv * acc


def _tc_d2_body(z2a_ref, z2b_ref, y2_ref, dinv_ref, b2_ref, w3_ref,
                y3_ref, hsum_ref):
    i = pl.program_id(0)
    dinv = dinv_ref[...]
    sub = lax.broadcasted_iota(jnp.int32, (_BR, LANE), 0)
    lane = lax.broadcasted_iota(jnp.int32, (_BR, LANE), 1)
    gid = (i * _BR + sub) * LANE + lane
    mask = (gid < N_NODES).astype(jnp.float32)

    @pl.when(i == 0)
    def _init():
        hsum_ref[...] = jnp.zeros((16, 1, LANE), jnp.float32)

    acc3 = None
    for f in range(16):
        h2 = jnp.maximum(
            dinv * (z2a_ref[f] + z2b_ref[f] + y2_ref[f]) + b2_ref[f], 0.0)
        hm = h2 * mask
        hsum_ref[f] = hsum_ref[f] + jnp.sum(hm, axis=0, keepdims=True)
        term = h2 * w3_ref[f, 0]
        acc3 = term if acc3 is None else acc3 + term
    y3_ref[...] = dinv * acc3


def _tc_e_body(z3a_ref, z3b_ref, y3_ref, dinv_ref, b3_ref,
               hsum_ref, wf_ref, bf_ref, choice_ref, value_ref):
    c = dinv_ref[...] * (z3a_ref[...] + z3b_ref[...] + y3_ref[...]) + b3_ref[0]
    sub = lax.broadcasted_iota(jnp.int32, (ROWS, LANE), 0)
    lane = lax.broadcasted_iota(jnp.int32, (ROWS, LANE), 1)
    valid = (sub * LANE + lane) < N_NODES
    cm = jnp.where(valid, c, -jnp.inf)
    m = jnp.max(cm)
    e = jnp.where(valid, jnp.exp(c - m), 0.0)
    s = jnp.sum(e)
    choice_ref[...] = e / s
    val = bf_ref[0]
    for f in range(16):
        val = val + jnp.sum(hsum_ref[f]) / float(N_NODES) * wf_ref[f, 0]
    value_ref[...] = jnp.full((1, LANE), val, jnp.float32)


def _tc_b(dega, degb, x0, x1):
    return pl.pallas_call(
        _tc_b_body,
        grid=(_GRID,),
        in_specs=[_rows_spec(1)] * 4,
        out_specs=[_rows_spec(1)] * 3,
        out_shape=[jax.ShapeDtypeStruct((ROWS, LANE), jnp.float32)] * 3,
    )(dega, degb, x0, x1)


def _tc_d1(z1a, z1b, y1, dinv, W1, b1, W2):
    return pl.pallas_call(
        _tc_d1_body,
        grid=(_GRID,),
        in_specs=[_rows_spec(2), _rows_spec(2), _rows_spec(2), _rows_spec(1),
                  _smem_spec(), _smem_spec(), _smem_spec()],
        out_specs=[_rows_spec(16)],
        out_shape=[jax.ShapeDtypeStruct((16, ROWS, LANE), jnp.float32)],
    )(z1a, z1b, y1, dinv, W1, b1, W2)[0]


def _tc_d2(z2a, z2b, y2, dinv, b2, W3):
    return pl.pallas_call(
        _tc_d2_body,
        grid=(_GRID,),
        in_specs=[_rows_spec(16), _rows_spec(16), _rows_spec(16),
                  _rows_spec(1), _smem_spec(), _smem_spec()],
        out_specs=[_rows_spec(1),
                   pl.BlockSpec((16, 1, LANE), lambda i: (0, 0, 0))],
        out_shape=[jax.ShapeDtypeStruct((ROWS, LANE), jnp.float32),
                   jax.ShapeDtypeStruct((16, 1, LANE), jnp.float32)],
    )(z2a, z2b, y2, dinv, b2, W3)


def _tc_e(z3a, z3b, y3, dinv, b3, hsum, Wf, bf):
    return pl.pallas_call(
        _tc_e_body,
        grid=(),
        in_specs=[pl.BlockSpec((ROWS, LANE), lambda: (0, 0))] * 4
        + [_smem_spec(), pl.BlockSpec((16, 1, LANE), lambda: (0, 0, 0)),
           _smem_spec(), _smem_spec()],
        out_specs=[pl.BlockSpec((ROWS, LANE), lambda: (0, 0)),
                   pl.BlockSpec((1, LANE), lambda: (0, 0))],
        out_shape=[jax.ShapeDtypeStruct((ROWS, LANE), jnp.float32),
                   jax.ShapeDtypeStruct((1, LANE), jnp.float32)],
    )(z3a, z3b, y3, dinv, b3, hsum, Wf, bf)


@jax.jit
def _run(x, edge_index, W1, b1, W2, b2, W3, b3, Wf, bf):
    f32 = jnp.float32
    pad_e = EP - E_EDGES
    ar = jnp.arange(pad_e, dtype=jnp.int32)
    src_rows = jnp.concatenate([edge_index[0], ar % N_NODES])
    dst_rows = jnp.concatenate(
        [edge_index[1], N_NODES + ar % (NP - N_NODES)])
    zeros1 = jnp.zeros((NP,), f32)

    xp = jnp.pad(x, ((0, NP - N_NODES), (0, 0)))
    x0 = xp[:, 0].reshape(ROWS, LANE)
    x1 = xp[:, 1].reshape(ROWS, LANE)

    deg = _get_sc_degree()(dst_rows, zeros1)
    dega = deg[0].reshape(ROWS, LANE)
    degb = deg[1].reshape(ROWS, LANE)
    dinv, y10, y11 = _tc_b(dega, degb, x0, x1)

    z1 = _make_prop_cols(2)(src_rows, dst_rows, zeros1,
                            y10.reshape(NP), y11.reshape(NP))
    z1a = jnp.stack([z1[0][0], z1[1][0]]).reshape(2, ROWS, LANE)
    z1b = jnp.stack([z1[0][1], z1[1][1]]).reshape(2, ROWS, LANE)
    y1 = jnp.stack([y10, y11])
    y2 = _tc_d1(z1a, z1b, y1, dinv, W1, b1, W2)

    z2 = _make_prop_cols(16)(src_rows, dst_rows, zeros1,
                             *[y2[f].reshape(NP) for f in range(16)])
    z2p = jnp.stack([z2[f] for f in range(16)], axis=1)
    z2p = z2p.reshape(2, 16, ROWS, LANE)
    y3, hsum = _tc_d2(z2p[0], z2p[1], y2, dinv, b2, W3)

    z3 = _make_prop_cols(1)(src_rows, dst_rows, zeros1,
                            y3.reshape(NP))[0]
    choice2d, value2d = _tc_e(
        z3[0].reshape(ROWS, LANE), z3[1].reshape(ROWS, LANE),
        y3, dinv, b3, hsum, Wf, bf)
    choice = choice2d.reshape(NP, 1)[:N_NODES]
    value = value2d[:1, :1]
    return choice, value


def kernel(x, edge_index, choices, W1, b1, W2, b2, W3, b3, Wf, bf):
    return _run(x, edge_index, W1, b1, W2, b2, W3, b3, Wf, bf)
